# baseline (speedup 1.0000x reference)
import functools

import jax
import jax.numpy as jnp
from jax import lax
from jax.experimental import pallas as pl
from jax.experimental.pallas import tpu as pltpu
from jax.experimental.pallas import tpu_sc as plsc

N = 4096
D_IN = 4096
H = 512
_EPS = 1e-5
_BLK = 256
_NBLK = N // _BLK
_NSLOT = 3
_DEPTH = 2


def _bn_relu(x, gamma, beta):
    mu = jnp.mean(x, axis=0, keepdims=True)
    xc = x - mu
    var = jnp.mean(xc * xc, axis=0, keepdims=True)
    y = gamma * xc * jax.lax.rsqrt(var + _EPS) + beta
    return jnp.maximum(y, 0.0)


_KPAD = 96


def _sc_extract(a):
    info = plsc.get_sparse_core_info()
    nw = info.num_cores * info.num_subcores
    rows_per = N // nw
    mesh = plsc.VectorSubcoreMesh(core_axis_name="c", subcore_axis_name="s")

    @functools.partial(
        pl.kernel, mesh=mesh,
        out_type=jax.ShapeDtypeStruct((N, _KPAD), jnp.int32),
        scratch_types=[
            pltpu.VMEM((N,), jnp.float32),
            pltpu.VMEM((_KPAD,), jnp.int32),
        ])
    def extract(a_hbm, cols_hbm, rowbuf, colbuf):
        wid = lax.axis_index("s") * info.num_cores + lax.axis_index("c")
        base = wid * rows_per

        def row_body(r, carry):
            row = base + r
            pltpu.sync_copy(a_hbm.at[row], rowbuf)
            for j in range(_KPAD // 16):
                colbuf[pl.ds(j * 16, 16)] = jnp.full((16,), N, jnp.int32)

            def chunk(c, cnt):
                v = rowbuf[pl.ds(c * 16, 16)]
                m = v != 0.0
                vals = jnp.arange(16, dtype=jnp.int32) + c * 16
                plsc.store_compressed(colbuf.at[pl.ds(cnt, 16)], vals, mask=m)
                pc = plsc.all_reduce_population_count(m)
                return jnp.minimum(cnt + pc[0], 80)

            cnt = lax.fori_loop(0, N // 16, chunk, 0)
            colbuf[pl.ds(80, 16)] = jnp.broadcast_to(cnt, (16,))
            pltpu.sync_copy(colbuf, cols_hbm.at[row])
            return carry

        lax.fori_loop(0, rows_per, row_body, 0)

    return extract(a)


def _gnn_kernel(f_hbm, a_hbm, w00_ref, w01_ref, w10_ref, w11_ref,
                g00_ref, be00_ref, g0_ref, be0_ref,
                g10_ref, be10_ref, g1_ref, be1_ref,
                nodes_ref, pool_ref,
                xv, hv, dbuf, sem):
    loads = ([(f_hbm, i) for i in range(_NBLK)] +
             [(a_hbm, i) for i in range(_NBLK)] +
             [(a_hbm, i) for i in range(_NBLK)])

    def copy(k, slot):
        src, blk = loads[k]
        return pltpu.make_async_copy(
            src.at[pl.ds(blk * _BLK, _BLK), :], dbuf.at[slot], sem.at[slot])

    for d in range(_DEPTH):
        copy(d, d % _NSLOT).start()
    for k in range(2 * _NBLK):
        slot = k % _NSLOT
        copy(k + _DEPTH, (k + _DEPTH) % _NSLOT).start()
        copy(k, slot).wait()
        rows = pl.ds(loads[k][1] * _BLK, _BLK)
        if k < _NBLK:
            xv[rows, :] = jnp.dot(dbuf[slot], w00_ref[...],
                                  preferred_element_type=jnp.float32)
        else:
            hv[rows, :] = jnp.dot(dbuf[slot], xv[...],
                                  preferred_element_type=jnp.float32)
    r = _bn_relu(hv[...], g00_ref[...], be00_ref[...])
    t = jnp.dot(r, w01_ref[...], preferred_element_type=jnp.float32)
    h = _bn_relu(t, g0_ref[...], be0_ref[...])
    xv[...] = jnp.dot(h, w10_ref[...], preferred_element_type=jnp.float32)
    for k in range(2 * _NBLK, 3 * _NBLK):
        slot = k % _NSLOT
        if k + _DEPTH < 3 * _NBLK:
            copy(k + _DEPTH, (k + _DEPTH) % _NSLOT).start()
        copy(k, slot).wait()
        rows = pl.ds(loads[k][1] * _BLK, _BLK)
        hv[rows, :] = jnp.dot(dbuf[slot], xv[...],
                              preferred_element_type=jnp.float32)
    r = _bn_relu(hv[...], g10_ref[...], be10_ref[...])
    t = jnp.dot(r, w11_ref[...], preferred_element_type=jnp.float32)
    out = _bn_relu(t, g1_ref[...], be1_ref[...])
    nodes_ref[...] = out
    pool_ref[...] = jnp.mean(out, axis=0, keepdims=True)


def kernel(features, adjacency_matrix, W0_0, b0_0, g0_0, be0_0, W0_1, b0_1,
           g0, be0, W1_0, b1_0, g1_0, be1_0, W1_1, b1_1, g1, be1):
    cols = _sc_extract(adjacency_matrix)
    features = lax.optimization_barrier((features, cols))[0]
    anyspec = pl.BlockSpec(memory_space=pl.ANY)
    full = lambda s: pl.BlockSpec(s, lambda: tuple(0 for _ in s))
    vec = full((1, H))
    nodes, pooled = pl.pallas_call(
        _gnn_kernel,
        in_specs=[anyspec, anyspec,
                  full((D_IN, H)), full((H, H)), full((H, H)), full((H, H)),
                  vec, vec, vec, vec, vec, vec, vec, vec],
        out_specs=[full((N, H)), full((1, H))],
        out_shape=[jax.ShapeDtypeStruct((N, H), jnp.float32),
                   jax.ShapeDtypeStruct((1, H), jnp.float32)],
        scratch_shapes=[
            pltpu.VMEM((N, H), jnp.float32),
            pltpu.VMEM((N, H), jnp.float32),
            pltpu.VMEM((_NSLOT, _BLK, D_IN), jnp.float32),
            pltpu.SemaphoreType.DMA((_NSLOT,)),
        ],
    )(features, adjacency_matrix, W0_0, W0_1, W1_0, W1_1,
      g0_0.reshape(1, H), be0_0.reshape(1, H), g0.reshape(1, H),
      be0.reshape(1, H), g1_0.reshape(1, H), be1_0.reshape(1, H),
      g1.reshape(1, H), be1.reshape(1, H))
    return (pooled, nodes)


# speedup vs baseline: 1.0017x; 1.0017x over previous
import jax
import jax.numpy as jnp
from jax.experimental import pallas as pl
from jax.experimental.pallas import tpu as pltpu

N = 4096
D_IN = 4096
H = 512
_EPS = 1e-5
_BLK = 256
_NBLK = N // _BLK
_NSLOT = 3
_DEPTH = 2


def _bn_relu(x, gamma, beta):
    mu = jnp.mean(x, axis=0, keepdims=True)
    xc = x - mu
    var = jnp.mean(xc * xc, axis=0, keepdims=True)
    y = gamma * xc * jax.lax.rsqrt(var + _EPS) + beta
    return jnp.maximum(y, 0.0)


def _gnn_kernel(f_hbm, a_hbm, w00_ref, w01_ref, w10_ref, w11_ref,
                g00_ref, be00_ref, g0_ref, be0_ref,
                g10_ref, be10_ref, g1_ref, be1_ref,
                nodes_ref, pool_ref,
                xv, hv, dbuf, sem):
    loads = ([(f_hbm, i) for i in range(_NBLK)] +
             [(a_hbm, i) for i in range(_NBLK)] +
             [(a_hbm, i) for i in range(_NBLK)])

    def copy(k, slot):
        src, blk = loads[k]
        return pltpu.make_async_copy(
            src.at[pl.ds(blk * _BLK, _BLK), :], dbuf.at[slot], sem.at[slot])

    for d in range(_DEPTH):
        copy(d, d % _NSLOT).start()
    for k in range(2 * _NBLK):
        slot = k % _NSLOT
        copy(k + _DEPTH, (k + _DEPTH) % _NSLOT).start()
        copy(k, slot).wait()
        rows = pl.ds(loads[k][1] * _BLK, _BLK)
        if k < _NBLK:
            xv[rows, :] = jnp.dot(dbuf[slot], w00_ref[...],
                                  preferred_element_type=jnp.float32)
        else:
            hv[rows, :] = jnp.dot(dbuf[slot], xv[...],
                                  preferred_element_type=jnp.float32)
    r = _bn_relu(hv[...], g00_ref[...], be00_ref[...])
    t = jnp.dot(r, w01_ref[...], preferred_element_type=jnp.float32)
    h = _bn_relu(t, g0_ref[...], be0_ref[...])
    xv[...] = jnp.dot(h, w10_ref[...], preferred_element_type=jnp.float32)
    for k in range(2 * _NBLK, 3 * _NBLK):
        slot = k % _NSLOT
        if k + _DEPTH < 3 * _NBLK:
            copy(k + _DEPTH, (k + _DEPTH) % _NSLOT).start()
        copy(k, slot).wait()
        rows = pl.ds(loads[k][1] * _BLK, _BLK)
        hv[rows, :] = jnp.dot(dbuf[slot], xv[...],
                              preferred_element_type=jnp.float32)
    r = _bn_relu(hv[...], g10_ref[...], be10_ref[...])
    t = jnp.dot(r, w11_ref[...], preferred_element_type=jnp.float32)
    out = _bn_relu(t, g1_ref[...], be1_ref[...])
    nodes_ref[...] = out
    pool_ref[...] = jnp.mean(out, axis=0, keepdims=True)


def kernel(features, adjacency_matrix, W0_0, b0_0, g0_0, be0_0, W0_1, b0_1,
           g0, be0, W1_0, b1_0, g1_0, be1_0, W1_1, b1_1, g1, be1):
    anyspec = pl.BlockSpec(memory_space=pl.ANY)
    full = lambda s: pl.BlockSpec(s, lambda: tuple(0 for _ in s))
    vec = full((1, H))
    nodes, pooled = pl.pallas_call(
        _gnn_kernel,
        in_specs=[anyspec, anyspec,
                  full((D_IN, H)), full((H, H)), full((H, H)), full((H, H)),
                  vec, vec, vec, vec, vec, vec, vec, vec],
        out_specs=[full((N, H)), full((1, H))],
        out_shape=[jax.ShapeDtypeStruct((N, H), jnp.float32),
                   jax.ShapeDtypeStruct((1, H), jnp.float32)],
        scratch_shapes=[
            pltpu.VMEM((N, H), jnp.float32),
            pltpu.VMEM((N, H), jnp.float32),
            pltpu.VMEM((_NSLOT, _BLK, D_IN), jnp.float32),
            pltpu.SemaphoreType.DMA((_NSLOT,)),
        ],
    )(features, adjacency_matrix, W0_0, W0_1, W1_0, W1_1,
      g0_0.reshape(1, H), be0_0.reshape(1, H), g0.reshape(1, H),
      be0.reshape(1, H), g1_0.reshape(1, H), be1_0.reshape(1, H),
      g1.reshape(1, H), be1.reshape(1, H))
    return (pooled, nodes)
